# baseline (speedup 1.0000x reference)
import functools

import jax
import jax.numpy as jnp
from jax import lax
from jax.experimental import pallas as pl
from jax.experimental.pallas import tpu as pltpu
from jax.experimental.pallas import tpu_sc as plsc

_E = 8
_K = 2
_T = 2048
_D = 2048
_F = 1024
_SCALE = 2.5
_BM = 256
_NT_CAP = (_K * _T + _E * (_BM - 1)) // _BM + 1
_CAP = _NT_CAP * _BM
_NP = _K * _T
_CHUNK = 128
_FC = 256
_NF = _F // _FC
_BT = 256

_INTERPRET = False


def _router_body(x_ref, gw_ref, gb_ref, dest_ref, wpair_ref, fe_ref,
                 fslot_ref, scomp_ref, used_ref, r_scratch):
    x = x_ref[...]
    logits = lax.dot_general(
        x.astype(jnp.bfloat16), gw_ref[...].astype(jnp.bfloat16),
        (((1,), (1,)), ((), ())), preferred_element_type=jnp.float32)
    scores = jax.nn.sigmoid(logits)
    biased = scores + gb_ref[...]

    ii = lax.broadcasted_iota(jnp.int32, (_T, _E), 1)
    m1 = jnp.max(biased, axis=1, keepdims=True)
    idx1 = jnp.min(jnp.where(biased == m1, ii, _E), axis=1, keepdims=True)
    b2 = jnp.where(ii == idx1, -jnp.inf, biased)
    m2 = jnp.max(b2, axis=1, keepdims=True)
    idx2 = jnp.min(jnp.where(b2 == m2, ii, _E), axis=1, keepdims=True)

    s1 = jnp.sum(jnp.where(ii == idx1, scores, 0.0), axis=1, keepdims=True)
    s2 = jnp.sum(jnp.where(ii == idx2, scores, 0.0), axis=1, keepdims=True)
    denom = s1 + s2 + 1e-20
    wpair_ref[0:_T, :] = s1 * _SCALE / denom
    wpair_ref[_T:_NP, :] = s2 * _SCALE / denom

    onehot1 = (ii == idx1).astype(jnp.float32)
    onehot2 = (ii == idx2).astype(jnp.float32)

    ci = lax.broadcasted_iota(jnp.int32, (_CHUNK, _CHUNK), 0)
    cj = lax.broadcasted_iota(jnp.int32, (_CHUNK, _CHUNK), 1)
    ltri = (ci > cj).astype(jnp.float32)

    tot = jnp.zeros((1, _E), jnp.float32)
    nh = _T // _CHUNK
    for c in range(2 * nh):
        oh = onehot1 if c < nh else onehot2
        base = (c % nh) * _CHUNK
        blk = oh[base:base + _CHUNK, :]
        r = lax.dot_general(ltri, blk, (((1,), (0,)), ((), ())),
                            preferred_element_type=jnp.float32) + tot
        r_scratch[c * _CHUNK:(c + 1) * _CHUNK, :] = r
        tot = tot + jnp.sum(blk, axis=0, keepdims=True)

    pc = jnp.floor((tot + (_BM - 1)) * (1.0 / _BM)) * _BM
    ei = lax.broadcasted_iota(jnp.int32, (_E, _E), 0)
    ej = lax.broadcasted_iota(jnp.int32, (_E, _E), 1)
    off = lax.dot_general(pc, (ei < ej).astype(jnp.float32),
                          (((1,), (0,)), ((), ())),
                          preferred_element_type=jnp.float32)

    for c in range(2 * nh):
        oh = onehot1 if c < nh else onehot2
        base = (c % nh) * _CHUNK
        blk = oh[base:base + _CHUNK, :]
        r = r_scratch[c * _CHUNK:(c + 1) * _CHUNK, :]
        d = jnp.sum(blk * (r + off), axis=1, keepdims=True)
        dest_ref[c * _CHUNK:(c + 1) * _CHUNK, :] = d.astype(jnp.int32)

    ti = lax.broadcasted_iota(jnp.int32, (_NT_CAP, _E), 0) * _BM
    tif = ti.astype(jnp.float32)
    te_f = jnp.sum((off <= tif).astype(jnp.float32), axis=1,
                   keepdims=True) - 1.0
    te_prev_f = jnp.sum((off <= tif - _BM).astype(jnp.float32), axis=1,
                        keepdims=True) - 1.0
    tidx = lax.broadcasted_iota(jnp.int32, (_NT_CAP, 1), 0)
    first = ((tidx == 0) | (te_f != te_prev_f)).astype(jnp.float32)
    used_ref[...] = (jnp.sum(pc, axis=1, keepdims=True) *
                     (1.0 / _BM)).astype(jnp.int32)

    oi = lax.broadcasted_iota(jnp.int32, (_NT_CAP, _NT_CAP), 0)
    oj = lax.broadcasted_iota(jnp.int32, (_NT_CAP, _NT_CAP), 1)
    ordf = lax.dot_general((oi >= oj).astype(jnp.float32), first,
                           (((1,), (0,)), ((), ())),
                           preferred_element_type=jnp.float32) - 1.0
    usede = (pc > 0.0).astype(jnp.float32)
    eordf = lax.dot_general(usede, (ei < ej).astype(jnp.float32),
                            (((1,), (0,)), ((), ())),
                            preferred_element_type=jnp.float32)
    nord = jnp.sum(usede, axis=1, keepdims=True)
    evec = lax.broadcasted_iota(jnp.int32, (1, _E), 1).astype(jnp.float32)
    match = (eordf == (ordf + 1.0)) & (usede > 0.0)
    nxte_f = jnp.sum(jnp.where(match, evec, 0.0), axis=1, keepdims=True)
    nxte_f = jnp.where((ordf + 1.0) < nord, nxte_f, te_f)

    fe_ref[0:1, :] = te_f[0:1, :].astype(jnp.int32)
    fe_ref[1:_NT_CAP + 1, :] = nxte_f.astype(jnp.int32)
    fslot_ref[0:1, :] = jnp.zeros((1, 1), jnp.int32)
    fslot_ref[1:_NT_CAP + 1, :] = (ordf.astype(jnp.int32) + 1) & 1
    scomp_ref[0:1, :] = jnp.zeros((1, 1), jnp.int32)
    scomp_ref[1:_NT_CAP + 1, :] = ordf.astype(jnp.int32) & 1


def _router(x, gate_w, gate_bias):
    return pl.pallas_call(
        _router_body,
        out_shape=[
            jax.ShapeDtypeStruct((_NP, 1), jnp.int32),
            jax.ShapeDtypeStruct((_NP, 1), jnp.float32),
            jax.ShapeDtypeStruct((_NT_CAP + 1, 1), jnp.int32),
            jax.ShapeDtypeStruct((_NT_CAP + 1, 1), jnp.int32),
            jax.ShapeDtypeStruct((_NT_CAP + 1, 1), jnp.int32),
            jax.ShapeDtypeStruct((1, 1), jnp.int32),
        ],
        scratch_shapes=[pltpu.VMEM((_NP, _E), jnp.float32)],
        interpret=_INTERPRET,
    )(x, gate_w, gate_bias.reshape(1, _E))


_NW = 32
_PPW = _NP // _NW
_CHA = 16
_NCA = _PPW // _CHA


def _dispatch_body(x_hbm, d_hbm, wp_hbm, xs_hbm, ws_hbm, dv, wv, rb0, rb1,
                   si0, si1, so0, so1):
    wid = lax.axis_index("s") * 2 + lax.axis_index("c")
    pltpu.sync_copy(d_hbm.at[wid], dv)
    pltpu.sync_copy(wp_hbm.at[wid], wv)
    for c in range(_NCA):
        pltpu.sync_copy(wv.at[c], ws_hbm.at[dv.at[c]])
    tb = pl.multiple_of((wid * _PPW) & (_T - 1), _PPW)
    rb = (rb0, rb1)
    sin = (si0, si1)
    sout = (so0, so1)
    din = [None, None]
    dout = [None, None]
    din[0] = pltpu.async_copy(x_hbm.at[pl.ds(tb, _CHA)], rb[0], sin[0])
    for c in range(_NCA):
        p = c % 2
        q = 1 - p
        if c + 1 < _NCA:
            if dout[q] is not None:
                dout[q].wait()
            din[q] = pltpu.async_copy(
                x_hbm.at[pl.ds(tb + (c + 1) * _CHA, _CHA)], rb[q], sin[q])
        din[p].wait()
        dout[p] = pltpu.async_copy(rb[p], xs_hbm.at[dv.at[c]], sout[p])
    dout[0].wait()
    dout[1].wait()


def _dispatch_gather(x, dest, wpair):
    d3 = dest.reshape(_NW, _NCA, _CHA)
    wp3 = wpair.reshape(_NW, _NCA, _CHA)
    mesh = plsc.VectorSubcoreMesh(core_axis_name="c", subcore_axis_name="s",
                                  num_cores=2, num_subcores=16)
    xs, ws = pl.kernel(
        _dispatch_body,
        out_type=[jax.ShapeDtypeStruct((_CAP, _D), jnp.float32),
                  jax.ShapeDtypeStruct((_CAP,), jnp.float32)],
        mesh=mesh,
        scratch_types=[
            pltpu.VMEM((_NCA, _CHA), jnp.int32),
            pltpu.VMEM((_NCA, _CHA), jnp.float32),
            pltpu.VMEM((_CHA, _D), jnp.float32),
            pltpu.VMEM((_CHA, _D), jnp.float32),
            pltpu.SemaphoreType.DMA,
            pltpu.SemaphoreType.DMA,
            pltpu.SemaphoreType.DMA,
            pltpu.SemaphoreType.DMA,
        ],
    )(x, d3, wp3)
    return xs, ws.reshape(_CAP, 1)


_NT_STEPS = _NT_CAP + 1


def _is_fetch_first(s, fe):
    return (s == 0) | (fe[s] != fe[jnp.maximum(s - 1, 0)])


def _ffn_tile(xsbf, w1bf, w3bf, w2bf, ws_ref, ys_ref):
    xs = xsbf[...]
    h = lax.dot_general(xs, w1bf[...], (((1,), (1,)), ((), ())),
                        preferred_element_type=jnp.float32)
    g = lax.dot_general(xs, w3bf[...], (((1,), (1,)), ((), ())),
                        preferred_element_type=jnp.float32)
    a = (h * jax.nn.sigmoid(h) * g).astype(jnp.bfloat16)
    o = lax.dot_general(a, w2bf[...], (((1,), (1,)), ((), ())),
                        preferred_element_type=jnp.float32)
    ys_ref[...] = o * ws_ref[...]


def _ffn_body(fe_ref, fslot_ref, scomp_ref, used_ref, xs_ref, w1_ref, w3_ref,
              w2_ref, ws_ref, ys_ref, w1bf0, w3bf0, w2bf0, w1bf1, w3bf1,
              w2bf1, xsbf):
    s = pl.program_id(0)
    f = pl.program_id(1)
    fetch_first = _is_fetch_first(s, fe_ref)

    def cast_into(w1bf, w3bf, w2bf):
        w1bf[pl.ds(f * _FC, _FC), :] = w1_ref[0].astype(jnp.bfloat16)
        w3bf[pl.ds(f * _FC, _FC), :] = w3_ref[0].astype(jnp.bfloat16)
        w2bf[:, pl.ds(f * _FC, _FC)] = w2_ref[0].astype(jnp.bfloat16)

    @pl.when(fetch_first & (fslot_ref[s] == 0))
    def _():
        cast_into(w1bf0, w3bf0, w2bf0)

    @pl.when(fetch_first & (fslot_ref[s] == 1))
    def _():
        cast_into(w1bf1, w3bf1, w2bf1)

    live = (s > 0) & (s - 1 < used_ref[0])

    @pl.when(live & (f == 0))
    def _():
        xsbf[...] = xs_ref[...].astype(jnp.bfloat16)

    @pl.when(live & (f == _NF - 1) & (scomp_ref[s] == 0))
    def _():
        _ffn_tile(xsbf, w1bf0, w3bf0, w2bf0, ws_ref, ys_ref)

    @pl.when(live & (f == _NF - 1) & (scomp_ref[s] == 1))
    def _():
        _ffn_tile(xsbf, w1bf1, w3bf1, w2bf1, ws_ref, ys_ref)


def _grouped_ffn(fe, fslot, scomp, used, xs, w1, w3, w2, ws):
    def wf_idx(s, f, fe):
        return jnp.where(_is_fetch_first(s, fe), f, 0)

    def t_idx(s):
        return jnp.maximum(s - 1, 0)

    grid_spec = pltpu.PrefetchScalarGridSpec(
        num_scalar_prefetch=4,
        grid=(_NT_STEPS, _NF),
        in_specs=[
            pl.BlockSpec((_BM, _D), lambda s, f, fe, fs, sc, u: (t_idx(s), 0)),
            pl.BlockSpec((1, _FC, _D),
                         lambda s, f, fe, fs, sc, u: (fe[s], wf_idx(s, f, fe),
                                                      0)),
            pl.BlockSpec((1, _FC, _D),
                         lambda s, f, fe, fs, sc, u: (fe[s], wf_idx(s, f, fe),
                                                      0)),
            pl.BlockSpec((1, _D, _FC),
                         lambda s, f, fe, fs, sc, u: (fe[s], 0,
                                                      wf_idx(s, f, fe))),
            pl.BlockSpec((_BM, 1), lambda s, f, fe, fs, sc, u: (t_idx(s), 0)),
        ],
        out_specs=pl.BlockSpec((_BM, _D),
                               lambda s, f, fe, fs, sc, u: (t_idx(s), 0)),
        scratch_shapes=[
            pltpu.VMEM((_F, _D), jnp.bfloat16),
            pltpu.VMEM((_F, _D), jnp.bfloat16),
            pltpu.VMEM((_D, _F), jnp.bfloat16),
            pltpu.VMEM((_F, _D), jnp.bfloat16),
            pltpu.VMEM((_F, _D), jnp.bfloat16),
            pltpu.VMEM((_D, _F), jnp.bfloat16),
            pltpu.VMEM((_BM, _D), jnp.bfloat16),
        ],
    )
    return pl.pallas_call(
        _ffn_body,
        grid_spec=grid_spec,
        out_shape=jax.ShapeDtypeStruct((_CAP, _D), jnp.float32),
        interpret=_INTERPRET,
    )(fe, fslot, scomp, used, xs, w1, w3, w2, ws)


def _shared_body(x_ref, w1_ref, w3_ref, w2_ref, out_ref, w1bf, w3bf, w2bf,
                 xbf):
    t = pl.program_id(0)
    f = pl.program_id(1)

    @pl.when(t == 0)
    def _():
        w1bf[pl.ds(f * _FC, _FC), :] = w1_ref[...].astype(jnp.bfloat16)
        w3bf[pl.ds(f * _FC, _FC), :] = w3_ref[...].astype(jnp.bfloat16)
        w2bf[:, pl.ds(f * _FC, _FC)] = w2_ref[...].astype(jnp.bfloat16)

    @pl.when(f == 0)
    def _():
        xbf[...] = x_ref[...].astype(jnp.bfloat16)

    @pl.when(f == _NF - 1)
    def _():
        x = xbf[...]
        h = lax.dot_general(x, w1bf[...], (((1,), (1,)), ((), ())),
                            preferred_element_type=jnp.float32)
        g = lax.dot_general(x, w3bf[...], (((1,), (1,)), ((), ())),
                            preferred_element_type=jnp.float32)
        a = (h * jax.nn.sigmoid(h) * g).astype(jnp.bfloat16)
        out_ref[...] = lax.dot_general(a, w2bf[...], (((1,), (1,)), ((), ())),
                                       preferred_element_type=jnp.float32)


def _shared_ffn(x, sw1, sw3, sw2):
    def wf(t, f):
        return jnp.where(t == 0, f, 0)

    return pl.pallas_call(
        _shared_body,
        grid=(_T // _BT, _NF),
        in_specs=[
            pl.BlockSpec((_BT, _D), lambda t, f: (t, 0)),
            pl.BlockSpec((_FC, _D), lambda t, f: (wf(t, f), 0)),
            pl.BlockSpec((_FC, _D), lambda t, f: (wf(t, f), 0)),
            pl.BlockSpec((_D, _FC), lambda t, f: (0, wf(t, f))),
        ],
        out_specs=pl.BlockSpec((_BT, _D), lambda t, f: (t, 0)),
        out_shape=jax.ShapeDtypeStruct((_T, _D), jnp.float32),
        scratch_shapes=[
            pltpu.VMEM((_F, _D), jnp.bfloat16),
            pltpu.VMEM((_F, _D), jnp.bfloat16),
            pltpu.VMEM((_D, _F), jnp.bfloat16),
            pltpu.VMEM((_BT, _D), jnp.bfloat16),
        ],
        interpret=_INTERPRET,
    )(x, sw1, sw3, sw2)


_TPW = _T // _NW
_CHB = 8
_NCB = _TPW // _CHB


def _combine_body(ys_hbm, d0_hbm, d1_hbm, sh_hbm, out_hbm, dv0, dv1,
                  bS0, bA0, bB0, bS1, bA1, bB1, si0, si1, so0, so1):
    wid = lax.axis_index("s") * 2 + lax.axis_index("c")
    pltpu.sync_copy(d0_hbm.at[wid], dv0)
    pltpu.sync_copy(d1_hbm.at[wid], dv1)
    tb = pl.multiple_of(wid * _TPW, _TPW)
    bufs = ((bS0, bA0, bB0), (bS1, bA1, bB1))
    sin = (si0, si1)
    sout = (so0, so1)

    def issue_in(c, p):
        bS, bA, bB = bufs[p]
        return (
            pltpu.async_copy(sh_hbm.at[pl.ds(tb + c * _CHB, _CHB)], bS,
                             sin[p]),
            pltpu.async_copy(ys_hbm.at[dv0.at[c]], bA, sin[p]),
            pltpu.async_copy(ys_hbm.at[dv1.at[c]], bB, sin[p]),
        )

    din = [None, None]
    dout = [None, None]
    din[0] = issue_in(0, 0)
    for c in range(_NCB):
        p = c % 2
        q = 1 - p
        if c + 1 < _NCB:
            if dout[q] is not None:
                dout[q].wait()
            din[q] = issue_in(c + 1, q)
        for d in din[p]:
            d.wait()
        bS, bA, bB = bufs[p]

        @pl.loop(0, _CHB)
        def _(r):
            @pl.loop(0, _D, step=16, unroll=8)
            def _(v):
                bS[r, pl.ds(v, 16)] = (bS[r, pl.ds(v, 16)] +
                                       bA[r, pl.ds(v, 16)] +
                                       bB[r, pl.ds(v, 16)])

        dout[p] = pltpu.async_copy(bS, out_hbm.at[pl.ds(tb + c * _CHB, _CHB)],
                                   sout[p])
    dout[0].wait()
    dout[1].wait()


def _combine(ys, dest, shared):
    d0 = dest[:_T].reshape(_NW, _NCB, _CHB)
    d1 = dest[_T:].reshape(_NW, _NCB, _CHB)
    mesh = plsc.VectorSubcoreMesh(core_axis_name="c", subcore_axis_name="s",
                                  num_cores=2, num_subcores=16)
    return pl.kernel(
        _combine_body,
        out_type=jax.ShapeDtypeStruct((_T, _D), jnp.float32),
        mesh=mesh,
        scratch_types=[
            pltpu.VMEM((_NCB, _CHB), jnp.int32),
            pltpu.VMEM((_NCB, _CHB), jnp.int32),
            pltpu.VMEM((_CHB, _D), jnp.float32),
            pltpu.VMEM((_CHB, _D), jnp.float32),
            pltpu.VMEM((_CHB, _D), jnp.float32),
            pltpu.VMEM((_CHB, _D), jnp.float32),
            pltpu.VMEM((_CHB, _D), jnp.float32),
            pltpu.VMEM((_CHB, _D), jnp.float32),
            pltpu.SemaphoreType.DMA,
            pltpu.SemaphoreType.DMA,
            pltpu.SemaphoreType.DMA,
            pltpu.SemaphoreType.DMA,
        ],
    )(ys, d0, d1, shared)


def kernel(x, gate_w, gate_bias, w1, w3, w2, sw1, sw3, sw2):
    dest, wpair, fe, fslot, scomp, used = _router(x, gate_w, gate_bias)
    dest = dest.reshape(_NP)
    xs, ws = _dispatch_gather(x, dest, wpair.reshape(_NP))
    shared = _shared_ffn(x, sw1, sw3, sw2)
    ys = _grouped_ffn(fe.reshape(_NT_STEPS), fslot.reshape(_NT_STEPS),
                      scomp.reshape(_NT_STEPS), used.reshape(1), xs,
                      w1, w3, w2, ws)
    return _combine(ys, dest, shared)
